# baseline (speedup 1.0000x reference)
import functools

import jax
import jax.numpy as jnp
from jax import lax
from jax.experimental import pallas as pl
from jax.experimental.pallas import tpu as pltpu
from jax.experimental.pallas import tpu_sc as plsc

B = 16384
D = 32
H1 = 64
H2 = 32

NC = 2
NS = 16
NW = NC * NS
BPW = B // NW


def _sc_gather(user_idx, item_idx, ue_gmf, ie_gmf, ue_mlp, ie_mlp):
  mesh = plsc.VectorSubcoreMesh(core_axis_name="c", subcore_axis_name="s")

  @functools.partial(
      pl.kernel,
      out_type=[jax.ShapeDtypeStruct((B, D), jnp.float32) for _ in range(4)],
      mesh=mesh,
      scratch_types=[
          pltpu.VMEM((BPW,), jnp.int32),
          pltpu.VMEM((BPW,), jnp.int32),
          pltpu.VMEM((BPW, D), jnp.float32),
          pltpu.VMEM((BPW, D), jnp.float32),
          pltpu.VMEM((BPW, D), jnp.float32),
          pltpu.VMEM((BPW, D), jnp.float32),
          pltpu.SemaphoreType.DMA,
      ],
      compiler_params=pltpu.CompilerParams(use_tc_tiling_on_sc=False),
  )
  def k(ui_hbm, ii_hbm, ug_hbm, ig_hbm, um_hbm, im_hbm,
        oug, oig, oum, oim,
        idx_u, idx_i, r_ug, r_ig, r_um, r_im, sem):
    wid = lax.axis_index("s") * NC + lax.axis_index("c")
    base = wid * BPW
    pltpu.sync_copy(ui_hbm.at[pl.ds(base, BPW)], idx_u)
    pltpu.sync_copy(ii_hbm.at[pl.ds(base, BPW)], idx_i)
    c1 = pltpu.async_copy(ug_hbm.at[idx_u], r_ug, sem)
    c2 = pltpu.async_copy(ig_hbm.at[idx_i], r_ig, sem)
    c3 = pltpu.async_copy(um_hbm.at[idx_u], r_um, sem)
    c4 = pltpu.async_copy(im_hbm.at[idx_i], r_im, sem)
    c1.wait()
    pltpu.sync_copy(r_ug, oug.at[pl.ds(base, BPW)])
    c2.wait()
    pltpu.sync_copy(r_ig, oig.at[pl.ds(base, BPW)])
    c3.wait()
    pltpu.sync_copy(r_um, oum.at[pl.ds(base, BPW)])
    c4.wait()
    pltpu.sync_copy(r_im, oim.at[pl.ds(base, BPW)])

  return k(user_idx, item_idx, ue_gmf, ie_gmf, ue_mlp, ie_mlp)


BLK = 2048


def _tc_mlp_body(ug, ig, um, im, w1u, w1i, b1, w2, b2, wog, wom, bo, out):
  h = um[...] @ w1u[...] + im[...] @ w1i[...] + b1[...]
  h = jnp.maximum(h, 0.0)
  h2 = jnp.maximum(h @ w2[...] + b2[...], 0.0)
  g = jnp.sum(ug[...] * ig[...] * wog[...], axis=1)
  logit = g + jnp.squeeze(h2 @ wom[...], axis=-1) + bo[0, 0]
  out[...] = jax.nn.sigmoid(logit)


def _tc_mlp(ug, ig, um, im, W1, b1, W2, b2, Wo, bo):
  w1u = W1[:, :D].T
  w1i = W1[:, D:].T
  w2 = W2.T
  wog = Wo[0, :D].reshape(1, D)
  wom = Wo[0, D:].reshape(H2, 1)
  b1r = b1.reshape(1, H1)
  b2r = b2.reshape(1, H2)
  bor = bo.reshape(1, 1)

  grid = (B // BLK,)
  blk2 = lambda d: pl.BlockSpec((BLK, d), lambda i: (i, 0))
  rep = lambda s: pl.BlockSpec(s, lambda i: (0,) * len(s))
  return pl.pallas_call(
      _tc_mlp_body,
      grid=grid,
      in_specs=[
          blk2(D), blk2(D), blk2(D), blk2(D),
          rep((D, H1)), rep((D, H1)), rep((1, H1)),
          rep((H1, H2)), rep((1, H2)),
          rep((1, D)), rep((H2, 1)), rep((1, 1)),
      ],
      out_specs=pl.BlockSpec((BLK,), lambda i: (i,)),
      out_shape=jax.ShapeDtypeStruct((B,), jnp.float32),
  )(ug, ig, um, im, w1u, w1i, b1r, w2, b2r, wog, wom, bor)


def kernel(user_idx, item_idx, ue_gmf, ie_gmf, ue_mlp, ie_mlp,
           W1, b1, W2, b2, Wo, bo):
  ui = user_idx.astype(jnp.int32)
  ii = item_idx.astype(jnp.int32)
  ug, ig, um, im = _sc_gather(ui, ii, ue_gmf, ie_gmf, ue_mlp, ie_mlp)
  return _tc_mlp(ug, ig, um, im, W1, b1, W2, b2, Wo, bo)
